# Initial kernel scaffold; baseline (speedup 1.0000x reference)
import jax
import jax.numpy as jnp
from jax.experimental import pallas as pl


def kernel(U, C, cat, U_table, C_table, cat_table0, cat_table1, cat_table2, cat_table3, W, b):
    raise NotImplementedError("write your pallas kernel here")


# speedup vs baseline: 11.6720x; 11.6720x over previous
import functools

import jax
import jax.numpy as jnp
from jax import lax
from jax.experimental import pallas as pl
from jax.experimental.pallas import tpu as pltpu
from jax.experimental.pallas import tpu_sc as plsc

_B = 16384
_L = 50
_NCAT = 4
_CVOC = 100000
_CATVOC = 10000
_UD = 32

_NC = 2
_NS = 16
_NW = _NC * _NS
_S = _B // _NW
_GRP = _S // 16
_CT_LEN = _NCAT * _CATVOC
_CAT_W = _L * _NCAT


def _project_tables(cflat, k0, k1, k2, k3, wcb, w0b, w1b, w2b, w3b):
    def body(c_ref, a0, a1, a2, a3, wc_ref, b0, b1, b2, b3,
             cs_out, o0, o1, o2, o3):
        cs_out[...] = jnp.dot(c_ref[...], wc_ref[...],
                              preferred_element_type=jnp.float32)
        for a, bb, o in ((a0, b0, o0), (a1, b1, o1), (a2, b2, o2), (a3, b3, o3)):
            o[...] = jnp.dot(a[...], bb[...],
                             preferred_element_type=jnp.float32)

    return pl.pallas_call(
        body,
        out_shape=[jax.ShapeDtypeStruct((3125, 32), jnp.float32)]
        + [jax.ShapeDtypeStruct((1250, 8), jnp.float32)] * 4,
    )(cflat, k0, k1, k2, k3, wcb, w0b, w1b, w2b, w3b)


def _sc_pooled_predict(pt, catidx, cidx, uidx, utab, params):
    mesh = plsc.VectorSubcoreMesh(core_axis_name="c", subcore_axis_name="s")

    @functools.partial(
        pl.kernel,
        out_type=jax.ShapeDtypeStruct((_B,), jnp.float32),
        mesh=mesh,
        scratch_types=[
            pltpu.VMEM((_CVOC,), jnp.float32),
            pltpu.VMEM((4, 128), jnp.int32),
            pltpu.VMEM((_S, _UD), jnp.float32),
            pltpu.VMEM((16 * _CAT_W,), jnp.int32),
            pltpu.VMEM((16 * _L,), jnp.int32),
            pltpu.VMEM((_S,), jnp.float32),
            pltpu.VMEM((40,), jnp.float32),
            pltpu.SemaphoreType.DMA,
        ],
        compiler_params=pltpu.CompilerParams(needs_layout_passes=False, use_tc_tiling_on_sc=False),
    )
    def k(pt_hbm, catidx_hbm, cidx_hbm, uidx_hbm, utab_hbm, par_hbm,
          out_hbm, tab_v, uidx_v, rows_v, cat_iv, c_iv, acc_v, par_v, usem):
        wid = lax.axis_index("s") * _NC + lax.axis_index("c")
        base = wid * _S

        for j in range(4):
            pltpu.sync_copy(uidx_hbm.at[pl.ds(base + j * 128, 128)],
                            uidx_v.at[j])
        cps = [
            pltpu.async_copy(utab_hbm.at[uidx_v.at[j]],
                             rows_v.at[pl.ds(j * 128, 128)], usem)
            for j in range(4)
        ]
        pltpu.sync_copy(par_hbm, par_v)

        lanes = lax.iota(jnp.int32, 16)

        pltpu.sync_copy(pt_hbm.at[pl.ds(0, _CT_LEN)],
                        tab_v.at[pl.ds(0, _CT_LEN)])
        cat_pos0 = lanes * _CAT_W

        def cat_group(g, carry):
            pltpu.sync_copy(
                catidx_hbm.at[pl.ds((base + g * 16) * _CAT_W, 16 * _CAT_W)],
                cat_iv)

            def step(t, acc):
                idx = plsc.load_gather(cat_iv, [cat_pos0 + t])
                idx = idx + jnp.bitwise_and(t, 3) * _CATVOC
                return acc + plsc.load_gather(tab_v, [idx])

            accv = lax.fori_loop(0, _CAT_W, step,
                                 jnp.zeros((16,), jnp.float32))
            acc_v[pl.ds(g * 16, 16)] = accv
            return carry

        lax.fori_loop(0, _GRP, cat_group, 0)

        for cp in cps:
            cp.wait()

        zeros16 = jnp.zeros((16,), jnp.int32)

        def u_group(g, carry):
            rid = g * 16 + lanes

            def ustep(d, acc):
                vals = plsc.load_gather(rows_v, [rid, zeros16 + d])
                w = plsc.load_gather(par_v, [zeros16 + d])
                return acc + vals * w

            accv = lax.fori_loop(0, _UD, ustep, acc_v[pl.ds(g * 16, 16)])
            acc_v[pl.ds(g * 16, 16)] = accv
            return carry

        lax.fori_loop(0, _GRP, u_group, 0)

        pltpu.sync_copy(pt_hbm.at[pl.ds(_CT_LEN, _CVOC)], tab_v)
        c_pos0 = lanes * _L
        bias_vec = plsc.load_gather(par_v, [zeros16 + 32])

        def c_group(g, carry):
            pltpu.sync_copy(
                cidx_hbm.at[pl.ds((base + g * 16) * _L, 16 * _L)], c_iv)

            def step(t, acc):
                idx = plsc.load_gather(c_iv, [c_pos0 + t])
                return acc + plsc.load_gather(tab_v, [idx])

            accv = lax.fori_loop(0, _L, step, acc_v[pl.ds(g * 16, 16)])
            z = accv + bias_vec
            acc_v[pl.ds(g * 16, 16)] = 1.0 / (1.0 + jnp.exp(-z))
            return carry

        lax.fori_loop(0, _GRP, c_group, 0)

        pltpu.sync_copy(acc_v, out_hbm.at[pl.ds(base, _S)])

    return k(pt, catidx, cidx, uidx, utab, params)


def kernel(U, C, cat, U_table, C_table, cat_table0, cat_table1, cat_table2,
           cat_table3, W, b):
    f32 = jnp.float32
    U = U.astype(jnp.int32)
    C = C.astype(jnp.int32)
    cat = cat.astype(jnp.int32)

    wc = W[0:32, 0]
    eye32 = jnp.eye(32, dtype=f32)
    wcb = (eye32[:, None, :] * wc[None, :, None]).reshape(1024, 32)
    eye8 = jnp.eye(8, dtype=f32)
    wibs = []
    for i in range(_NCAT):
        wi = W[32 + 16 * i:48 + 16 * i, 0]
        wibs.append((eye8[:, None, :] * wi[None, :, None]).reshape(128, 8))

    cflat = C_table.reshape(3125, 1024)
    kflats = [t.reshape(1250, 128)
              for t in (cat_table0, cat_table1, cat_table2, cat_table3)]

    cs, o0, o1, o2, o3 = _project_tables(cflat, *kflats, wcb, *wibs)
    pt = jnp.concatenate([o0.reshape(-1), o1.reshape(-1), o2.reshape(-1),
                          o3.reshape(-1), cs.reshape(-1)])
    params = jnp.concatenate([W[96:128, 0], b, jnp.zeros((7,), f32)])

    out = _sc_pooled_predict(pt, cat.reshape(-1), C.reshape(-1), U,
                             U_table, params)
    return out.reshape(_B, 1)
